# Initial kernel scaffold; baseline (speedup 1.0000x reference)
import jax
import jax.numpy as jnp
from jax.experimental import pallas as pl


def kernel(x, edge_index, edge_attr, W1, b1, Wq, bq, Wk, bk, Wv, bv, We, Wskip, bskip, gamma, beta):
    raise NotImplementedError("write your pallas kernel here")


# speedup vs baseline: 1.3410x; 1.3410x over previous
import functools

import jax
import jax.numpy as jnp
from jax import lax
from jax.experimental import pallas as pl
from jax.experimental.pallas import tpu as pltpu

N = 10000
E = 320000
D_IN = 128
D_E = 16
D_EMB = 128
D_H = 2 * D_EMB


def _matmul_bias_kernel(x_ref, w_ref, b_ref, o_ref):
    o_ref[...] = (
        jnp.dot(x_ref[...], w_ref[...], preferred_element_type=jnp.float32)
        + b_ref[...]
    )


def _matmul_bias(x, w, b, block_rows=1000):
    n, d_in = x.shape
    d_out = w.shape[1]
    return pl.pallas_call(
        _matmul_bias_kernel,
        grid=(n // block_rows,),
        in_specs=[
            pl.BlockSpec((block_rows, d_in), lambda i: (i, 0)),
            pl.BlockSpec((d_in, d_out), lambda i: (0, 0)),
            pl.BlockSpec((1, d_out), lambda i: (0, 0)),
        ],
        out_specs=pl.BlockSpec((block_rows, d_out), lambda i: (i, 0)),
        out_shape=jax.ShapeDtypeStruct((n, d_out), jnp.float32),
    )(x, w, b.reshape(1, -1))


def kernel(x, edge_index, edge_attr, W1, b1, Wq, bq, Wk, bk, Wv, bv, We,
           Wskip, bskip, gamma, beta):
    n = x.shape[0]
    src = edge_index[0].astype(jnp.int32)
    dst = edge_index[1].astype(jnp.int32)

    deg = jax.ops.segment_sum(jnp.ones((E,), jnp.float32), dst, num_segments=n) + 1.0
    dis = lax.rsqrt(jnp.maximum(deg, 1.0))
    xw = _matmul_bias(x, W1, jnp.zeros((D_H,), jnp.float32))
    norm = dis[src] * dis[dst]
    h = jax.ops.segment_sum(norm[:, None] * xw[src], dst, num_segments=n)
    h = h + (dis * dis)[:, None] * xw + b1
    h = jax.nn.relu(h)

    q = _matmul_bias(h, Wq, bq)
    k = _matmul_bias(h, Wk, bk)
    v = _matmul_bias(h, Wv, bv)
    qe = q @ We.T

    inv_sqrt = 1.0 / jnp.sqrt(float(D_EMB))
    logits = (jnp.sum(q[dst] * k[src], axis=-1)
              + jnp.sum(qe[dst] * edge_attr, axis=-1)) * inv_sqrt
    p = jnp.exp(logits)
    z = jax.ops.segment_sum(p, dst, num_segments=n)
    alpha = p / jnp.maximum(z[dst], 1e-16)
    out = jax.ops.segment_sum(alpha[:, None] * v[src], dst, num_segments=n)
    acc_attr = jax.ops.segment_sum(alpha[:, None] * edge_attr, dst, num_segments=n)
    out = out + acc_attr @ We + _matmul_bias(h, Wskip, bskip)

    mu = jnp.mean(out, axis=-1, keepdims=True)
    var = jnp.mean((out - mu) ** 2, axis=-1, keepdims=True)
    out = (out - mu) * lax.rsqrt(var + 1e-5) * gamma + beta
    out = jax.nn.relu(out)
    return jnp.mean(out, axis=0, keepdims=True)


# speedup vs baseline: 2.8076x; 2.0937x over previous
import functools

import jax
import jax.numpy as jnp
from jax import lax
from jax.experimental import pallas as pl
from jax.experimental.pallas import tpu as pltpu
from jax.experimental.pallas import tpu_sc as plsc

N = 10000
E = 320000
D_IN = 128
D_E = 16
D_EMB = 128
D_H = 2 * D_EMB
DP = 2 * D_EMB

NC = 2
NS = 16
L = 16
NW = NC * NS
NP = 10240
HALF = N // 2
NPH = 5120
NPHT = NPH // NS
TRASH = NPH - 1
QUARTER = N // 4
NPQ = 2560
NPQT = NPQ // NS
TRASHQ = NPQ - 1
CB = 80

E_PAD = 327680
PAD_DST = NP - 1
EPT_B = E_PAD // NS
NCH_B = EPT_B // CB
EPT = E_PAD // NW
NCH = EPT // CB
_INV_SQRT = 1.0 / float(D_EMB) ** 0.5

_SC_MESH = plsc.VectorSubcoreMesh(core_axis_name="c", subcore_axis_name="s")
_SC_PARAMS = pltpu.CompilerParams(needs_layout_passes=False)


def _zero_vec(ref, nwords):
    def body(i, _):
        ref[pl.ds(i * L, L)] = jnp.zeros((L,), jnp.float32)
        return 0
    lax.fori_loop(0, nwords // L, body, 0)


def _zero_rows(ref, nrows, ncols):
    def body(i, _):
        for t in range(ncols // L):
            ref[i, pl.ds(t * L, L)] = jnp.zeros((L,), jnp.float32)
        return 0
    lax.fori_loop(0, nrows, body, 0)


def _localize_dst(dix, d2all, base, extent, trash, nchunks):
    def body(j, _):
        for g in range(CB // L):
            d16 = dix[pl.ds(j * CB + g * L, L)]
            da = d16 - base
            ok = (da >= 0) & (da < extent)
            d2all[j, pl.ds(g * L, L)] = jnp.where(ok, da, trash)
        return 0
    lax.fori_loop(0, nchunks, body, 0)


def _deg_body(dst_hbm, deg_hbm, idx_v, deg_v):
    c = lax.axis_index("c")
    s = lax.axis_index("s")
    wid = c * NS + s
    pltpu.sync_copy(dst_hbm.at[pl.ds(wid * EPT, EPT)], idx_v)
    _zero_vec(deg_v, NP)
    ones = jnp.full((L,), 1.0, jnp.float32)

    def body(i, _):
        idx16 = idx_v[pl.ds(i * L, L)]
        plsc.addupdate_scatter(deg_v, [idx16], ones)
        return 0
    lax.fori_loop(0, EPT // L, body, 0)
    pltpu.sync_copy(deg_v, deg_hbm.at[pl.ds(wid * NP, NP)])


_deg_pass = functools.partial(
    pl.kernel,
    out_type=jax.ShapeDtypeStruct((NW * NP,), jnp.float32),
    mesh=_SC_MESH,
    compiler_params=_SC_PARAMS,
    scratch_types=[
        pltpu.VMEM((EPT,), jnp.int32),
        pltpu.VMEM((NP,), jnp.float32),
    ],
)(_deg_body)


def _make_gcn_pass(qi):
    def _gcn_body(src_hbm, dst_hbm, x_hbm, dis_hbm, xacc_hbm,
                  six, dix, d2all, dis_v, rows, sem, xacc):
        c = lax.axis_index("c")
        s = lax.axis_index("s")
        pltpu.sync_copy(src_hbm.at[pl.ds(s * EPT_B, EPT_B)], six)
        pltpu.sync_copy(dst_hbm.at[pl.ds(s * EPT_B, EPT_B)], dix)
        pltpu.sync_copy(dis_hbm, dis_v)
        _localize_dst(dix, d2all, (2 * qi + c) * QUARTER, QUARTER, TRASHQ,
                      NCH_B)

        _zero_rows(rows, CB, D_IN)
        for kk in range(NPQT // CB):
            pltpu.sync_copy(rows, xacc.at[pl.ds(s * NPQT + kk * CB, CB)])
        plsc.subcore_barrier()

        def body(j, _):
            pltpu.async_copy(x_hbm.at[six.at[pl.ds(j * CB, CB)]], rows,
                             sem).wait()
            for g in range(CB // L):
                s16 = six[pl.ds(j * CB + g * L, L)]
                d16 = dix[pl.ds(j * CB + g * L, L)]
                n16 = (plsc.load_gather(dis_v, [s16])
                       * plsc.load_gather(dis_v, [d16]))
                for e in range(L):
                    ne = jnp.full((L,), n16[e], jnp.float32)
                    r = g * L + e
                    for t in range(D_IN // L):
                        rows[r, pl.ds(t * L, L)] = (
                            rows[r, pl.ds(t * L, L)] * ne)
            pltpu.async_copy(rows, xacc.at[d2all.at[j]], sem,
                             add=True).wait()
            return 0

        lax.fori_loop(0, NCH_B, body, 0)
        plsc.subcore_barrier()
        pltpu.sync_copy(xacc.at[pl.ds(s * NPQT, NPQT)],
                        xacc_hbm.at[pl.ds(c * NPQ + s * NPQT, NPQT)])

    return functools.partial(
        pl.kernel,
        out_type=jax.ShapeDtypeStruct((NC * NPQ, D_IN), jnp.float32),
        mesh=_SC_MESH,
        compiler_params=_SC_PARAMS,
        scratch_types=[
            pltpu.VMEM((EPT_B,), jnp.int32),
            pltpu.VMEM((EPT_B,), jnp.int32),
            pltpu.VMEM((NCH_B, CB), jnp.int32),
            pltpu.VMEM((NP,), jnp.float32),
            pltpu.VMEM((CB, D_IN), jnp.float32),
            pltpu.SemaphoreType.DMA,
            pltpu.VMEM_SHARED((NPQ, D_IN), jnp.float32),
        ],
    )(_gcn_body)


_gcn_pass0 = _make_gcn_pass(0)
_gcn_pass1 = _make_gcn_pass(1)


def _logit_body(src_hbm, dst_hbm, qp_hbm, k_hbm, attr3_hbm,
                p_hbm, z_hbm,
                six, dix, qrows, krows, a16v, pflat, z_v, sem):
    c = lax.axis_index("c")
    s = lax.axis_index("s")
    wid = c * NS + s
    pltpu.sync_copy(src_hbm.at[pl.ds(wid * EPT, EPT)], six)
    pltpu.sync_copy(dst_hbm.at[pl.ds(wid * EPT, EPT)], dix)
    _zero_vec(z_v, NP)

    def body(j, _):
        pltpu.async_copy(qp_hbm.at[dix.at[pl.ds(j * CB, CB)]], qrows,
                         sem).wait()
        pltpu.async_copy(k_hbm.at[six.at[pl.ds(j * CB, CB)]], krows,
                         sem).wait()
        pltpu.sync_copy(attr3_hbm.at[wid * NCH + j], a16v)
        for g in range(CB // L):
            lvec = jnp.zeros((L,), jnp.float32)
            for e in range(L):
                r = g * L + e
                arow = a16v[r // 8, pl.ds((r % 8) * D_E, D_E)]
                acc = qrows[r, pl.ds(D_EMB, L)] * arow
                for t in range(D_EMB // L):
                    acc = acc + (qrows[r, pl.ds(t * L, L)]
                                 * krows[r, pl.ds(t * L, L)])
                lsc = jnp.sum(acc)
                lvec = jnp.where(lax.iota(jnp.int32, L) == e,
                                 jnp.full((L,), lsc, jnp.float32), lvec)
            pv = jnp.exp(lvec * _INV_SQRT)
            pflat[pl.ds(j * CB + g * L, L)] = pv
            d16 = dix[pl.ds(j * CB + g * L, L)]
            plsc.addupdate_scatter(z_v, [d16], pv)
        return 0

    lax.fori_loop(0, NCH, body, 0)
    pltpu.sync_copy(pflat, p_hbm.at[pl.ds(wid * EPT, EPT)])
    pltpu.sync_copy(z_v, z_hbm.at[pl.ds(wid * NP, NP)])


_logit_pass = functools.partial(
    pl.kernel,
    out_type=(jax.ShapeDtypeStruct((E_PAD,), jnp.float32),
              jax.ShapeDtypeStruct((NW * NP,), jnp.float32)),
    mesh=_SC_MESH,
    compiler_params=_SC_PARAMS,
    scratch_types=[
        pltpu.VMEM((EPT,), jnp.int32),
        pltpu.VMEM((EPT,), jnp.int32),
        pltpu.VMEM((CB, DP), jnp.float32),
        pltpu.VMEM((CB, D_EMB), jnp.float32),
        pltpu.VMEM((CB // 8, 8 * D_E), jnp.float32),
        pltpu.VMEM((EPT,), jnp.float32),
        pltpu.VMEM((NP,), jnp.float32),
        pltpu.SemaphoreType.DMA,
    ],
)(_logit_body)


def _pattr_body(dst_hbm, attr3_hbm, p_hbm, pa_hbm, dix, pflat, a16v, histo):
    c = lax.axis_index("c")
    s = lax.axis_index("s")
    wid = c * NS + s
    base = c * HALF
    pltpu.sync_copy(dst_hbm.at[pl.ds(s * EPT_B, EPT_B)], dix)
    pltpu.sync_copy(p_hbm.at[pl.ds(s * EPT_B, EPT_B)], pflat)
    _zero_vec(histo, NPH * D_E)
    lanes = lax.iota(jnp.int32, L)

    def body(j, _):
        pltpu.sync_copy(attr3_hbm.at[s * NCH_B + j], a16v)
        for g in range(CB // L):
            d16 = dix[pl.ds(j * CB + g * L, L)]
            da = d16 - base
            ok = (da >= 0) & (da < HALF)
            dl = jnp.where(ok, da, TRASH)
            p16 = pflat[pl.ds(j * CB + g * L, L)]
            for e in range(L):
                r = g * L + e
                flat = jnp.full((L,), dl[e] * D_E, jnp.int32) + lanes
                val = (a16v[r // 8, pl.ds((r % 8) * D_E, D_E)]
                       * jnp.full((L,), p16[e], jnp.float32))
                plsc.addupdate_scatter(histo, [flat], val)
        return 0

    lax.fori_loop(0, NCH_B, body, 0)
    pltpu.sync_copy(histo, pa_hbm.at[pl.ds(wid * NPH * D_E, NPH * D_E)])


_pattr_pass = functools.partial(
    pl.kernel,
    out_type=jax.ShapeDtypeStruct((NW * NPH * D_E,), jnp.float32),
    mesh=_SC_MESH,
    compiler_params=_SC_PARAMS,
    scratch_types=[
        pltpu.VMEM((EPT_B,), jnp.int32),
        pltpu.VMEM((EPT_B,), jnp.float32),
        pltpu.VMEM((CB // 8, 8 * D_E), jnp.float32),
        pltpu.VMEM((NPH * D_E,), jnp.float32),
    ],
)(_pattr_body)


def _make_agg_pass(qi):
    def _agg_body(src_hbm, dst_hbm, v_hbm, p_hbm, z_hbm, out_hbm,
                  six, dix, d2all, z_v, vrows, pflat, sem, outacc):
        c = lax.axis_index("c")
        s = lax.axis_index("s")
        pltpu.sync_copy(src_hbm.at[pl.ds(s * EPT_B, EPT_B)], six)
        pltpu.sync_copy(dst_hbm.at[pl.ds(s * EPT_B, EPT_B)], dix)
        pltpu.sync_copy(z_hbm.at[pl.ds((2 * qi + c) * NPQ, NPQ)], z_v)
        pltpu.sync_copy(p_hbm.at[pl.ds(s * EPT_B, EPT_B)], pflat)
        _localize_dst(dix, d2all, (2 * qi + c) * QUARTER, QUARTER, TRASHQ,
                      NCH_B)

        _zero_rows(vrows, CB, D_EMB)
        for kk in range(NPQT // CB):
            pltpu.sync_copy(vrows, outacc.at[pl.ds(s * NPQT + kk * CB, CB)])
        plsc.subcore_barrier()

        eps = jnp.full((L,), 1e-16, jnp.float32)

        def body(j, _):
            pltpu.async_copy(v_hbm.at[six.at[pl.ds(j * CB, CB)]], vrows,
                             sem).wait()
            for g in range(CB // L):
                d16 = d2all[j, pl.ds(g * L, L)]
                zl = plsc.load_gather(z_v, [d16])
                al = pflat[pl.ds(j * CB + g * L, L)] / jnp.maximum(zl, eps)
                for e in range(L):
                    ab = jnp.full((L,), al[e], jnp.float32)
                    r = g * L + e
                    for t in range(D_EMB // L):
                        vrows[r, pl.ds(t * L, L)] = (
                            vrows[r, pl.ds(t * L, L)] * ab)
            pltpu.async_copy(vrows, outacc.at[d2all.at[j]], sem,
                             add=True).wait()
            return 0

        lax.fori_loop(0, NCH_B, body, 0)
        plsc.subcore_barrier()
        pltpu.sync_copy(outacc.at[pl.ds(s * NPQT, NPQT)],
                        out_hbm.at[pl.ds(c * NPQ + s * NPQT, NPQT)])

    return functools.partial(
        pl.kernel,
        out_type=jax.ShapeDtypeStruct((NC * NPQ, D_EMB), jnp.float32),
        mesh=_SC_MESH,
        compiler_params=_SC_PARAMS,
        scratch_types=[
            pltpu.VMEM((EPT_B,), jnp.int32),
            pltpu.VMEM((EPT_B,), jnp.int32),
            pltpu.VMEM((NCH_B, CB), jnp.int32),
            pltpu.VMEM((NPQ,), jnp.float32),
            pltpu.VMEM((CB, D_EMB), jnp.float32),
            pltpu.VMEM((EPT_B,), jnp.float32),
            pltpu.SemaphoreType.DMA,
            pltpu.VMEM_SHARED((NPQ, D_EMB), jnp.float32),
        ],
    )(_agg_body)


_agg_pass0 = _make_agg_pass(0)
_agg_pass1 = _make_agg_pass(1)


def _dis_kernel(dp_ref, dis_ref):
    deg = jnp.sum(dp_ref[...], axis=0, keepdims=True) + 1.0
    dis_ref[...] = lax.rsqrt(deg)


def _dis_pass(deg_parts):
    return pl.pallas_call(
        _dis_kernel,
        out_shape=jax.ShapeDtypeStruct((1, NP), jnp.float32),
    )(deg_parts)


def _zsum_kernel(zp_ref, z_ref):
    z_ref[...] = jnp.sum(zp_ref[...], axis=0, keepdims=True)


def _zsum_pass(z_parts):
    return pl.pallas_call(
        _zsum_kernel,
        out_shape=jax.ShapeDtypeStruct((1, NP), jnp.float32),
    )(z_parts)


def _pasum_kernel(pa_ref, o_ref):
    o_ref[...] = jnp.sum(pa_ref[...], axis=1)


def _pasum_pass(pa):
    return pl.pallas_call(
        _pasum_kernel,
        out_shape=jax.ShapeDtypeStruct((NC, NPH * D_E), jnp.float32),
    )(pa)


def _enc_kernel(xa_ref, x_ref, dis_ref, w1_ref, b1_ref,
                wq_ref, bq_ref, wk_ref, bk_ref, wv_ref, bv_ref, we_ref,
                h_ref, qp_ref, k_ref, v_ref):
    dis = dis_ref[...]
    xin = xa_ref[...] + (dis * dis) * x_ref[...]
    h = jnp.dot(xin, w1_ref[...], preferred_element_type=jnp.float32) + b1_ref[...]
    h = jnp.maximum(h, 0.0)
    h_ref[...] = h
    q = jnp.dot(h, wq_ref[...], preferred_element_type=jnp.float32) + bq_ref[...]
    qe = lax.dot_general(q, we_ref[...], (((1,), (1,)), ((), ())),
                         preferred_element_type=jnp.float32)
    rows = q.shape[0]
    qp_ref[...] = jnp.concatenate(
        [q, qe, jnp.zeros((rows, DP - D_EMB - D_E), jnp.float32)], axis=1)
    k_ref[...] = jnp.dot(h, wk_ref[...], preferred_element_type=jnp.float32) + bk_ref[...]
    v_ref[...] = jnp.dot(h, wv_ref[...], preferred_element_type=jnp.float32) + bv_ref[...]


def _enc_pass(xa, x, dis_n, W1, b1, Wq, bq, Wk, bk, Wv, bv, We,
              block_rows=1000):
    nblocks = N // block_rows
    full = lambda shape: pl.BlockSpec(shape, lambda i: tuple(0 for _ in shape))
    row_spec = lambda d: pl.BlockSpec((block_rows, d), lambda i: (i, 0))
    return pl.pallas_call(
        _enc_kernel,
        grid=(nblocks,),
        in_specs=[
            row_spec(D_IN), row_spec(D_IN), row_spec(1),
            full((D_IN, D_H)), full((1, D_H)),
            full((D_H, D_EMB)), full((1, D_EMB)),
            full((D_H, D_EMB)), full((1, D_EMB)),
            full((D_H, D_EMB)), full((1, D_EMB)),
            full((D_E, D_EMB)),
        ],
        out_specs=[row_spec(D_H), row_spec(DP), row_spec(D_EMB),
                   row_spec(D_EMB)],
        out_shape=[
            jax.ShapeDtypeStruct((N, D_H), jnp.float32),
            jax.ShapeDtypeStruct((N, DP), jnp.float32),
            jax.ShapeDtypeStruct((N, D_EMB), jnp.float32),
            jax.ShapeDtypeStruct((N, D_EMB), jnp.float32),
        ],
    )(xa, x, dis_n, W1, b1.reshape(1, -1), Wq, bq.reshape(1, -1),
      Wk, bk.reshape(1, -1), Wv, bv.reshape(1, -1), We)


def _final_kernel(out_ref, ap_ref, z_ref, h_ref, we_ref,
                  wskip_ref, bskip_ref, gamma_ref, beta_ref,
                  o_ref, acc_ref):
    i = pl.program_id(0)
    ap = ap_ref[...] / jnp.maximum(z_ref[...], 1e-16)
    out = (out_ref[...]
           + jnp.dot(ap, we_ref[...],
                     preferred_element_type=jnp.float32)
           + jnp.dot(h_ref[...], wskip_ref[...],
                     preferred_element_type=jnp.float32)
           + bskip_ref[...])
    mu = jnp.mean(out, axis=-1, keepdims=True)
    var = jnp.mean((out - mu) ** 2, axis=-1, keepdims=True)
    out = (out - mu) * lax.rsqrt(var + 1e-5) * gamma_ref[...] + beta_ref[...]
    out = jnp.maximum(out, 0.0)
    part = jnp.sum(out, axis=0, keepdims=True)

    @pl.when(i == 0)
    def _():
        acc_ref[...] = jnp.zeros_like(acc_ref)

    acc_ref[...] += part

    @pl.when(i == pl.num_programs(0) - 1)
    def _():
        o_ref[...] = acc_ref[...] * (1.0 / N)


def _final_pass(out_sc, ap, zn, h, We, Wskip, bskip, gamma, beta,
                block_rows=1000):
    nblocks = N // block_rows
    full = lambda shape: pl.BlockSpec(shape, lambda i: tuple(0 for _ in shape))
    row_spec = lambda d: pl.BlockSpec((block_rows, d), lambda i: (i, 0))
    return pl.pallas_call(
        _final_kernel,
        grid=(nblocks,),
        in_specs=[
            row_spec(D_EMB), row_spec(D_E), row_spec(1), row_spec(D_H),
            full((D_E, D_EMB)), full((D_H, D_EMB)), full((1, D_EMB)),
            full((1, D_EMB)), full((1, D_EMB)),
        ],
        out_specs=full((1, D_EMB)),
        out_shape=jax.ShapeDtypeStruct((1, D_EMB), jnp.float32),
        scratch_shapes=[pltpu.VMEM((1, D_EMB), jnp.float32)],
    )(out_sc, ap, zn, h, We, Wskip, bskip.reshape(1, -1),
      gamma.reshape(1, -1), beta.reshape(1, -1))


def kernel(x, edge_index, edge_attr, W1, b1, Wq, bq, Wk, bk, Wv, bv, We,
           Wskip, bskip, gamma, beta):
    npad = E_PAD - E
    src = jnp.pad(edge_index[0].astype(jnp.int32), (0, npad))
    dst = jnp.pad(edge_index[1].astype(jnp.int32), (0, npad),
                  constant_values=PAD_DST)
    attr3 = jnp.pad(edge_attr, ((0, npad), (0, 0))).reshape(
        E_PAD // CB, CB // 8, 8 * D_E)

    deg_parts = _deg_pass(dst).reshape(NW, NP)
    dis = _dis_pass(deg_parts).reshape(NP)
    xq0 = _gcn_pass0(src, dst, x, dis)
    xq1 = _gcn_pass1(src, dst, x, dis)
    xacc = jnp.concatenate([xq0[:QUARTER], xq0[NPQ:NPQ + QUARTER],
                            xq1[:QUARTER], xq1[NPQ:NPQ + QUARTER]], 0)
    h, qp, k, v = _enc_pass(xacc, x, dis[:N].reshape(N, 1),
                            W1, b1, Wq, bq, Wk, bk, Wv, bv, We)

    p, z_parts = _logit_pass(src, dst, qp, k, attr3)
    z = _zsum_pass(z_parts.reshape(NW, NP)).reshape(NP)
    zq = jnp.pad(z[:N].reshape(4, QUARTER),
                 ((0, 0), (0, NPQ - QUARTER))).reshape(4 * NPQ)
    oq0 = _agg_pass0(src, dst, v, p, zq)
    oq1 = _agg_pass1(src, dst, v, p, zq)
    out_sc = jnp.concatenate([oq0[:QUARTER], oq0[NPQ:NPQ + QUARTER],
                              oq1[:QUARTER], oq1[NPQ:NPQ + QUARTER]], 0)
    pa = _pasum_pass(_pattr_pass(dst, attr3, p).reshape(NC, NS, NPH * D_E))
    pa2 = pa.reshape(NC, NPH, D_E)
    ap = jnp.concatenate([pa2[0, :HALF], pa2[1, :HALF]], 0)

    return _final_pass(out_sc, ap, z[:N].reshape(N, 1), h,
                       We, Wskip, bskip, gamma, beta)


# speedup vs baseline: 4.1486x; 1.4776x over previous
import functools

import jax
import jax.numpy as jnp
from jax import lax
from jax.experimental import pallas as pl
from jax.experimental.pallas import tpu as pltpu
from jax.experimental.pallas import tpu_sc as plsc

N = 10000
E = 320000
D_IN = 128
D_E = 16
D_EMB = 128
D_H = 2 * D_EMB
DP = 2 * D_EMB

NC = 2
NS = 16
L = 16
NW = NC * NS
NP = 10240
HALF = N // 2
NPH = 5120
NPHT = NPH // NS
TRASH = NPH - 1
QUARTER = N // 4
NPQ = 2560
NPQT = NPQ // NS
TRASHQ = NPQ - 1
CB = 80

E_PAD = 327680
PAD_DST = NP - 1
EPT_B = E_PAD // NS
NCH_B = EPT_B // CB
EPT = E_PAD // NW
NCH = EPT // CB
_INV_SQRT = 1.0 / float(D_EMB) ** 0.5

_SC_MESH = plsc.VectorSubcoreMesh(core_axis_name="c", subcore_axis_name="s")
_SC_PARAMS = pltpu.CompilerParams(needs_layout_passes=False)


def _zero_vec(ref, nwords):
    def body(i, _):
        ref[pl.ds(i * L, L)] = jnp.zeros((L,), jnp.float32)
        return 0
    lax.fori_loop(0, nwords // L, body, 0)


def _zero_rows(ref, nrows, ncols):
    def body(i, _):
        for t in range(ncols // L):
            ref[i, pl.ds(t * L, L)] = jnp.zeros((L,), jnp.float32)
        return 0
    lax.fori_loop(0, nrows, body, 0)


def _localize_dst(dix, d2all, base, extent, trash, nchunks):
    def body(j, _):
        for g in range(CB // L):
            d16 = dix[pl.ds(j * CB + g * L, L)]
            da = d16 - base
            ok = (da >= 0) & (da < extent)
            d2all[j, pl.ds(g * L, L)] = jnp.where(ok, da, trash)
        return 0
    lax.fori_loop(0, nchunks, body, 0)


def _deg_body(dst_hbm, deg_hbm, idx_v, deg_v):
    c = lax.axis_index("c")
    s = lax.axis_index("s")
    wid = c * NS + s
    pltpu.sync_copy(dst_hbm.at[pl.ds(wid * EPT, EPT)], idx_v)
    _zero_vec(deg_v, NP)
    ones = jnp.full((L,), 1.0, jnp.float32)

    def body(i, _):
        idx16 = idx_v[pl.ds(i * L, L)]
        plsc.addupdate_scatter(deg_v, [idx16], ones)
        return 0
    lax.fori_loop(0, EPT // L, body, 0)
    pltpu.sync_copy(deg_v, deg_hbm.at[pl.ds(wid * NP, NP)])


_deg_pass = functools.partial(
    pl.kernel,
    out_type=jax.ShapeDtypeStruct((NW * NP,), jnp.float32),
    mesh=_SC_MESH,
    compiler_params=_SC_PARAMS,
    scratch_types=[
        pltpu.VMEM((EPT,), jnp.int32),
        pltpu.VMEM((NP,), jnp.float32),
    ],
)(_deg_body)


EPT_H = EPT_B // 2
NCH_H = NCH_B // 2


def _gcn_body(src_hbm, dst_hbm, x_hbm, dis_hbm, xacc_hbm,
              six, dix, d2all, dis_v, rows, sem, xacc):
    c = lax.axis_index("c")
    s = lax.axis_index("s")
    pltpu.sync_copy(dis_hbm, dis_v)

    _zero_rows(rows, CB, D_IN)
    for kk in range(NPHT // CB):
        pltpu.sync_copy(rows, xacc.at[pl.ds(s * NPHT + kk * CB, CB)])
    plsc.subcore_barrier()

    def body(j, _):
        pltpu.async_copy(x_hbm.at[six.at[pl.ds(j * CB, CB)]], rows,
                         sem).wait()
        for g in range(CB // L):
            s16 = six[pl.ds(j * CB + g * L, L)]
            d16 = dix[pl.ds(j * CB + g * L, L)]
            n16 = (plsc.load_gather(dis_v, [s16])
                   * plsc.load_gather(dis_v, [d16]))
            for e in range(L):
                ne = jnp.full((L,), n16[e], jnp.float32)
                r = g * L + e
                for t in range(D_IN // L):
                    rows[r, pl.ds(t * L, L)] = rows[r, pl.ds(t * L, L)] * ne
        pltpu.async_copy(rows, xacc.at[d2all.at[j]], sem, add=True).wait()
        return 0

    for half in range(2):
        off = s * EPT_B + half * EPT_H
        pltpu.sync_copy(src_hbm.at[pl.ds(off, EPT_H)], six)
        pltpu.sync_copy(dst_hbm.at[pl.ds(off, EPT_H)], dix)
        _localize_dst(dix, d2all, c * HALF, HALF, TRASH, NCH_H)
        lax.fori_loop(0, NCH_H, body, 0)

    plsc.subcore_barrier()
    pltpu.sync_copy(xacc.at[pl.ds(s * NPHT, NPHT)],
                    xacc_hbm.at[pl.ds(c * NPH + s * NPHT, NPHT)])


_gcn_pass = functools.partial(
    pl.kernel,
    out_type=jax.ShapeDtypeStruct((NC * NPH, D_IN), jnp.float32),
    mesh=_SC_MESH,
    compiler_params=_SC_PARAMS,
    scratch_types=[
        pltpu.VMEM((EPT_H,), jnp.int32),
        pltpu.VMEM((EPT_H,), jnp.int32),
        pltpu.VMEM((NCH_H, CB), jnp.int32),
        pltpu.VMEM((NP,), jnp.float32),
        pltpu.VMEM((CB, D_IN), jnp.float32),
        pltpu.SemaphoreType.DMA,
        pltpu.VMEM_SHARED((NPH, D_IN), jnp.float32),
    ],
)(_gcn_body)


def _logit_body(src_hbm, dst_hbm, qp_hbm, k_hbm, attr3_hbm,
                p_hbm, z_hbm,
                six, dix, qrows, krows, a16v, pflat, z_v, sem):
    c = lax.axis_index("c")
    s = lax.axis_index("s")
    wid = c * NS + s
    pltpu.sync_copy(src_hbm.at[pl.ds(wid * EPT, EPT)], six)
    pltpu.sync_copy(dst_hbm.at[pl.ds(wid * EPT, EPT)], dix)
    _zero_vec(z_v, NP)

    def body(j, _):
        pltpu.async_copy(qp_hbm.at[dix.at[pl.ds(j * CB, CB)]], qrows,
                         sem).wait()
        pltpu.async_copy(k_hbm.at[six.at[pl.ds(j * CB, CB)]], krows,
                         sem).wait()
        pltpu.sync_copy(attr3_hbm.at[wid * NCH + j], a16v)
        for g in range(CB // L):
            lvec = jnp.zeros((L,), jnp.float32)
            for e in range(L):
                r = g * L + e
                arow = a16v[r // 8, pl.ds((r % 8) * D_E, D_E)]
                acc = qrows[r, pl.ds(D_EMB, L)] * arow
                for t in range(D_EMB // L):
                    acc = acc + (qrows[r, pl.ds(t * L, L)]
                                 * krows[r, pl.ds(t * L, L)])
                lsc = jnp.sum(acc)
                lvec = jnp.where(lax.iota(jnp.int32, L) == e,
                                 jnp.full((L,), lsc, jnp.float32), lvec)
            pv = jnp.exp(lvec * _INV_SQRT)
            pflat[pl.ds(j * CB + g * L, L)] = pv
            d16 = dix[pl.ds(j * CB + g * L, L)]
            plsc.addupdate_scatter(z_v, [d16], pv)
        return 0

    lax.fori_loop(0, NCH, body, 0)
    pltpu.sync_copy(pflat, p_hbm.at[pl.ds(wid * EPT, EPT)])
    pltpu.sync_copy(z_v, z_hbm.at[pl.ds(wid * NP, NP)])


_logit_pass = functools.partial(
    pl.kernel,
    out_type=(jax.ShapeDtypeStruct((E_PAD,), jnp.float32),
              jax.ShapeDtypeStruct((NW * NP,), jnp.float32)),
    mesh=_SC_MESH,
    compiler_params=_SC_PARAMS,
    scratch_types=[
        pltpu.VMEM((EPT,), jnp.int32),
        pltpu.VMEM((EPT,), jnp.int32),
        pltpu.VMEM((CB, DP), jnp.float32),
        pltpu.VMEM((CB, D_EMB), jnp.float32),
        pltpu.VMEM((CB // 8, 8 * D_E), jnp.float32),
        pltpu.VMEM((EPT,), jnp.float32),
        pltpu.VMEM((NP,), jnp.float32),
        pltpu.SemaphoreType.DMA,
    ],
)(_logit_body)


def _pattr_body(dst_hbm, attr3_hbm, p_hbm, pa_hbm, dix, pflat, a16v, histo):
    c = lax.axis_index("c")
    s = lax.axis_index("s")
    wid = c * NS + s
    base = c * HALF
    pltpu.sync_copy(dst_hbm.at[pl.ds(s * EPT_B, EPT_B)], dix)
    pltpu.sync_copy(p_hbm.at[pl.ds(s * EPT_B, EPT_B)], pflat)
    _zero_vec(histo, NPH * D_E)
    lanes = lax.iota(jnp.int32, L)

    def body(j, _):
        pltpu.sync_copy(attr3_hbm.at[s * NCH_B + j], a16v)
        for g in range(CB // L):
            d16 = dix[pl.ds(j * CB + g * L, L)]
            da = d16 - base
            ok = (da >= 0) & (da < HALF)
            dl = jnp.where(ok, da, TRASH)
            p16 = pflat[pl.ds(j * CB + g * L, L)]
            for e in range(L):
                r = g * L + e
                flat = jnp.full((L,), dl[e] * D_E, jnp.int32) + lanes
                val = (a16v[r // 8, pl.ds((r % 8) * D_E, D_E)]
                       * jnp.full((L,), p16[e], jnp.float32))
                plsc.addupdate_scatter(histo, [flat], val)
        return 0

    lax.fori_loop(0, NCH_B, body, 0)
    pltpu.sync_copy(histo, pa_hbm.at[pl.ds(wid * NPH * D_E, NPH * D_E)])


_pattr_pass = functools.partial(
    pl.kernel,
    out_type=jax.ShapeDtypeStruct((NW * NPH * D_E,), jnp.float32),
    mesh=_SC_MESH,
    compiler_params=_SC_PARAMS,
    scratch_types=[
        pltpu.VMEM((EPT_B,), jnp.int32),
        pltpu.VMEM((EPT_B,), jnp.float32),
        pltpu.VMEM((CB // 8, 8 * D_E), jnp.float32),
        pltpu.VMEM((NPH * D_E,), jnp.float32),
    ],
)(_pattr_body)


def _agg_body(src_hbm, dst_hbm, v_hbm, p_hbm, z_hbm, out_hbm,
              six, dix, d2all, z_v, vrows, pflat, sem, outacc):
    c = lax.axis_index("c")
    s = lax.axis_index("s")
    pltpu.sync_copy(z_hbm.at[pl.ds(c * NPH, NPH)], z_v)

    _zero_rows(vrows, CB, D_EMB)
    for kk in range(NPHT // CB):
        pltpu.sync_copy(vrows, outacc.at[pl.ds(s * NPHT + kk * CB, CB)])
    plsc.subcore_barrier()

    eps = jnp.full((L,), 1e-16, jnp.float32)

    def body(j, _):
        pltpu.async_copy(v_hbm.at[six.at[pl.ds(j * CB, CB)]], vrows,
                         sem).wait()
        for g in range(CB // L):
            d16 = d2all[j, pl.ds(g * L, L)]
            zl = plsc.load_gather(z_v, [d16])
            al = pflat[pl.ds(j * CB + g * L, L)] / jnp.maximum(zl, eps)
            for e in range(L):
                ab = jnp.full((L,), al[e], jnp.float32)
                r = g * L + e
                for t in range(D_EMB // L):
                    vrows[r, pl.ds(t * L, L)] = vrows[r, pl.ds(t * L, L)] * ab
        pltpu.async_copy(vrows, outacc.at[d2all.at[j]], sem,
                         add=True).wait()
        return 0

    for half in range(2):
        off = s * EPT_B + half * EPT_H
        pltpu.sync_copy(src_hbm.at[pl.ds(off, EPT_H)], six)
        pltpu.sync_copy(dst_hbm.at[pl.ds(off, EPT_H)], dix)
        pltpu.sync_copy(p_hbm.at[pl.ds(off, EPT_H)], pflat)
        _localize_dst(dix, d2all, c * HALF, HALF, TRASH, NCH_H)
        lax.fori_loop(0, NCH_H, body, 0)

    plsc.subcore_barrier()
    pltpu.sync_copy(outacc.at[pl.ds(s * NPHT, NPHT)],
                    out_hbm.at[pl.ds(c * NPH + s * NPHT, NPHT)])


_agg_pass = functools.partial(
    pl.kernel,
    out_type=jax.ShapeDtypeStruct((NC * NPH, D_EMB), jnp.float32),
    mesh=_SC_MESH,
    compiler_params=_SC_PARAMS,
    scratch_types=[
        pltpu.VMEM((EPT_H,), jnp.int32),
        pltpu.VMEM((EPT_H,), jnp.int32),
        pltpu.VMEM((NCH_H, CB), jnp.int32),
        pltpu.VMEM((NPH,), jnp.float32),
        pltpu.VMEM((CB, D_EMB), jnp.float32),
        pltpu.VMEM((EPT_H,), jnp.float32),
        pltpu.SemaphoreType.DMA,
        pltpu.VMEM_SHARED((NPH, D_EMB), jnp.float32),
    ],
)(_agg_body)


def _dis_kernel(dp_ref, dis_ref):
    deg = jnp.sum(dp_ref[...], axis=0, keepdims=True) + 1.0
    dis_ref[...] = lax.rsqrt(deg)


def _dis_pass(deg_parts):
    return pl.pallas_call(
        _dis_kernel,
        out_shape=jax.ShapeDtypeStruct((1, NP), jnp.float32),
    )(deg_parts)


def _zsum_kernel(zp_ref, z_ref):
    z_ref[...] = jnp.sum(zp_ref[...], axis=0, keepdims=True)


def _zsum_pass(z_parts):
    return pl.pallas_call(
        _zsum_kernel,
        out_shape=jax.ShapeDtypeStruct((1, NP), jnp.float32),
    )(z_parts)


def _pasum_kernel(pa_ref, o_ref):
    o_ref[...] = jnp.sum(pa_ref[...], axis=1)


def _pasum_pass(pa):
    return pl.pallas_call(
        _pasum_kernel,
        out_shape=jax.ShapeDtypeStruct((NC, NPH * D_E), jnp.float32),
    )(pa)


def _enc_kernel(xa_ref, x_ref, dis_ref, w1_ref, b1_ref,
                wq_ref, bq_ref, wk_ref, bk_ref, wv_ref, bv_ref, we_ref,
                h_ref, qp_ref, k_ref, v_ref):
    dis = dis_ref[...]
    xin = xa_ref[...] + (dis * dis) * x_ref[...]
    h = jnp.dot(xin, w1_ref[...], preferred_element_type=jnp.float32) + b1_ref[...]
    h = jnp.maximum(h, 0.0)
    h_ref[...] = h
    q = jnp.dot(h, wq_ref[...], preferred_element_type=jnp.float32) + bq_ref[...]
    qe = lax.dot_general(q, we_ref[...], (((1,), (1,)), ((), ())),
                         preferred_element_type=jnp.float32)
    rows = q.shape[0]
    qp_ref[...] = jnp.concatenate(
        [q, qe, jnp.zeros((rows, DP - D_EMB - D_E), jnp.float32)], axis=1)
    k_ref[...] = jnp.dot(h, wk_ref[...], preferred_element_type=jnp.float32) + bk_ref[...]
    v_ref[...] = jnp.dot(h, wv_ref[...], preferred_element_type=jnp.float32) + bv_ref[...]


def _enc_pass(xa, x, dis_n, W1, b1, Wq, bq, Wk, bk, Wv, bv, We,
              block_rows=1000):
    nblocks = N // block_rows
    full = lambda shape: pl.BlockSpec(shape, lambda i: tuple(0 for _ in shape))
    row_spec = lambda d: pl.BlockSpec((block_rows, d), lambda i: (i, 0))
    return pl.pallas_call(
        _enc_kernel,
        grid=(nblocks,),
        in_specs=[
            row_spec(D_IN), row_spec(D_IN), row_spec(1),
            full((D_IN, D_H)), full((1, D_H)),
            full((D_H, D_EMB)), full((1, D_EMB)),
            full((D_H, D_EMB)), full((1, D_EMB)),
            full((D_H, D_EMB)), full((1, D_EMB)),
            full((D_E, D_EMB)),
        ],
        out_specs=[row_spec(D_H), row_spec(DP), row_spec(D_EMB),
                   row_spec(D_EMB)],
        out_shape=[
            jax.ShapeDtypeStruct((N, D_H), jnp.float32),
            jax.ShapeDtypeStruct((N, DP), jnp.float32),
            jax.ShapeDtypeStruct((N, D_EMB), jnp.float32),
            jax.ShapeDtypeStruct((N, D_EMB), jnp.float32),
        ],
    )(xa, x, dis_n, W1, b1.reshape(1, -1), Wq, bq.reshape(1, -1),
      Wk, bk.reshape(1, -1), Wv, bv.reshape(1, -1), We)


def _final_kernel(out_ref, ap_ref, z_ref, h_ref, we_ref,
                  wskip_ref, bskip_ref, gamma_ref, beta_ref,
                  o_ref, acc_ref):
    i = pl.program_id(0)
    ap = ap_ref[...] / jnp.maximum(z_ref[...], 1e-16)
    out = (out_ref[...]
           + jnp.dot(ap, we_ref[...],
                     preferred_element_type=jnp.float32)
           + jnp.dot(h_ref[...], wskip_ref[...],
                     preferred_element_type=jnp.float32)
           + bskip_ref[...])
    mu = jnp.mean(out, axis=-1, keepdims=True)
    var = jnp.mean((out - mu) ** 2, axis=-1, keepdims=True)
    out = (out - mu) * lax.rsqrt(var + 1e-5) * gamma_ref[...] + beta_ref[...]
    out = jnp.maximum(out, 0.0)
    part = jnp.sum(out, axis=0, keepdims=True)

    @pl.when(i == 0)
    def _():
        acc_ref[...] = jnp.zeros_like(acc_ref)

    acc_ref[...] += part

    @pl.when(i == pl.num_programs(0) - 1)
    def _():
        o_ref[...] = acc_ref[...] * (1.0 / N)


def _final_pass(out_sc, ap, zn, h, We, Wskip, bskip, gamma, beta,
                block_rows=1000):
    nblocks = N // block_rows
    full = lambda shape: pl.BlockSpec(shape, lambda i: tuple(0 for _ in shape))
    row_spec = lambda d: pl.BlockSpec((block_rows, d), lambda i: (i, 0))
    return pl.pallas_call(
        _final_kernel,
        grid=(nblocks,),
        in_specs=[
            row_spec(D_EMB), row_spec(D_E), row_spec(1), row_spec(D_H),
            full((D_E, D_EMB)), full((D_H, D_EMB)), full((1, D_EMB)),
            full((1, D_EMB)), full((1, D_EMB)),
        ],
        out_specs=full((1, D_EMB)),
        out_shape=jax.ShapeDtypeStruct((1, D_EMB), jnp.float32),
        scratch_shapes=[pltpu.VMEM((1, D_EMB), jnp.float32)],
    )(out_sc, ap, zn, h, We, Wskip, bskip.reshape(1, -1),
      gamma.reshape(1, -1), beta.reshape(1, -1))


def kernel(x, edge_index, edge_attr, W1, b1, Wq, bq, Wk, bk, Wv, bv, We,
           Wskip, bskip, gamma, beta):
    npad = E_PAD - E
    src = jnp.pad(edge_index[0].astype(jnp.int32), (0, npad))
    dst = jnp.pad(edge_index[1].astype(jnp.int32), (0, npad),
                  constant_values=PAD_DST)
    attr3 = jnp.pad(edge_attr, ((0, npad), (0, 0))).reshape(
        E_PAD // CB, CB // 8, 8 * D_E)

    deg_parts = _deg_pass(dst).reshape(NW, NP)
    dis = _dis_pass(deg_parts).reshape(NP)
    xh = _gcn_pass(src, dst, x, dis)
    xacc = jnp.concatenate([xh[:HALF], xh[NPH:NPH + HALF]], 0)
    h, qp, k, v = _enc_pass(xacc, x, dis[:N].reshape(N, 1),
                            W1, b1, Wq, bq, Wk, bk, Wv, bv, We)

    p, z_parts = _logit_pass(src, dst, qp, k, attr3)
    z = _zsum_pass(z_parts.reshape(NW, NP)).reshape(NP)
    zq = jnp.pad(z[:N].reshape(2, HALF),
                 ((0, 0), (0, NPH - HALF))).reshape(2 * NPH)
    oh = _agg_pass(src, dst, v, p, zq)
    out_sc = jnp.concatenate([oh[:HALF], oh[NPH:NPH + HALF]], 0)
    pa = _pasum_pass(_pattr_pass(dst, attr3, p).reshape(NC, NS, NPH * D_E))
    pa2 = pa.reshape(NC, NPH, D_E)
    ap = jnp.concatenate([pa2[0, :HALF], pa2[1, :HALF]], 0)

    return _final_pass(out_sc, ap, z[:N].reshape(N, 1), h,
                       We, Wskip, bskip, gamma, beta)
